# baseline (speedup 1.0000x reference)
import functools

import jax
import jax.numpy as jnp
from jax import lax
from jax.experimental import pallas as pl
from jax.experimental.pallas import tpu as pltpu

B, S, DIM = 4, 4096, 64
N_HASHES = 4
BUCKET_SIZE = 64
N_BUCKETS = S // BUCKET_SIZE
N_CHUNKS = N_HASHES * N_BUCKETS
CHUNK = (N_HASHES * S) // N_CHUNKS
TOKEN_SELF_ATTN_VALUE = -5e4
NEG_BIG = 2**30


def _hash_body(qk_ref, rot_ref, buckets_ref):
    x = qk_ref[0]
    r = jnp.dot(x, rot_ref[...], preferred_element_type=jnp.float32)
    iota = lax.broadcasted_iota(jnp.int32, (S, 2 * (N_BUCKETS // 2)), 1)
    for h in range(N_HASHES):
        seg = r[:, h * (N_BUCKETS // 2):(h + 1) * (N_BUCKETS // 2)]
        full = jnp.concatenate([seg, -seg], axis=1)
        m = jnp.max(full, axis=1, keepdims=True)
        idx = jnp.min(jnp.where(full == m, iota, NEG_BIG), axis=1)
        buckets_ref[0, h] = idx + h * N_BUCKETS


def _hash_buckets(qk, rot128):
    return pl.pallas_call(
        _hash_body,
        grid=(B,),
        in_specs=[
            pl.BlockSpec((1, S, DIM), lambda b: (b, 0, 0)),
            pl.BlockSpec((DIM, N_HASHES * (N_BUCKETS // 2)), lambda b: (0, 0)),
        ],
        out_specs=pl.BlockSpec((1, N_HASHES, S), lambda b: (b, 0, 0)),
        out_shape=jax.ShapeDtypeStruct((B, N_HASHES, S), jnp.int32),
    )(qk, rot128)


def _attn_body(q_ref, kprev_ref, vself_ref, vprev_ref, tq_ref, tprev_ref,
               so_ref, slse_ref):
    q = q_ref[0, 0]
    kp = kprev_ref[0, 0]

    def norm(x):
        n = jnp.sqrt(jnp.sum(x * x, axis=1, keepdims=True))
        return x / jnp.maximum(n, 1e-12)

    ks = norm(q)
    kpn = norm(kp)
    tq = tq_ref[0, 0, 0]
    tp = tprev_ref[0, 0, 0]

    scale = DIM ** -0.5
    d_self = lax.dot_general(q, ks, (((1,), (1,)), ((), ()))) * scale
    d_prev = lax.dot_general(q, kpn, (((1,), (1,)), ((), ()))) * scale
    d_self = jnp.where(tq[:, None] == tq[None, :], TOKEN_SELF_ATTN_VALUE, d_self)
    d_prev = jnp.where(tq[:, None] == tp[None, :], TOKEN_SELF_ATTN_VALUE, d_prev)
    dots = jnp.concatenate([d_self, d_prev], axis=1)

    m = jnp.max(dots, axis=1, keepdims=True)
    e = jnp.exp(dots - m)
    lse = m + jnp.log(jnp.sum(e, axis=1, keepdims=True))
    p = jnp.exp(dots - lse)
    vcat = jnp.concatenate([vself_ref[0, 0], vprev_ref[0, 0]], axis=0)
    so_ref[0, 0] = jnp.dot(p, vcat, preferred_element_type=jnp.float32)
    slse_ref[0, 0, 0] = lse[:, 0]


def _attention(sqk4, sv4, st4):
    qb = pl.BlockSpec((1, 1, CHUNK, DIM), lambda b, c: (b, c, 0, 0))
    pb = pl.BlockSpec((1, 1, CHUNK, DIM),
                      lambda b, c: (b, (c - 1) % N_CHUNKS, 0, 0))
    tb = pl.BlockSpec((1, 1, 1, CHUNK), lambda b, c: (b, c, 0, 0))
    tpb = pl.BlockSpec((1, 1, 1, CHUNK),
                       lambda b, c: (b, (c - 1) % N_CHUNKS, 0, 0))
    return pl.pallas_call(
        _attn_body,
        grid=(B, N_CHUNKS),
        in_specs=[qb, pb, qb, pb, tb, tpb],
        out_specs=[qb, tb],
        out_shape=[
            jax.ShapeDtypeStruct((B, N_CHUNKS, CHUNK, DIM), jnp.float32),
            jax.ShapeDtypeStruct((B, N_CHUNKS, 1, CHUNK), jnp.float32),
        ],
    )(sqk4, sqk4, sv4, sv4, st4, st4)


def _combine_body(lg_ref, o_ref, out_ref):
    lg = lg_ref[0]
    m = jnp.max(lg, axis=0, keepdims=True)
    lse = m + jnp.log(jnp.sum(jnp.exp(lg - m), axis=0, keepdims=True))
    w = jnp.exp(lg - lse)
    o = o_ref[0]
    out_ref[0] = jnp.sum(o * w[:, :, None], axis=0)


def _combine(logits, o):
    TBLK = 512
    return pl.pallas_call(
        _combine_body,
        grid=(B, S // TBLK),
        in_specs=[
            pl.BlockSpec((1, N_HASHES, TBLK), lambda b, t: (b, 0, t)),
            pl.BlockSpec((1, N_HASHES, TBLK, DIM), lambda b, t: (b, 0, t, 0)),
        ],
        out_specs=pl.BlockSpec((1, TBLK, DIM), lambda b, t: (b, t, 0)),
        out_shape=jax.ShapeDtypeStruct((B, S, DIM), jnp.float32),
    )(logits, o)


def kernel(qk, v, random_rotations):
    rot128 = random_rotations[0].reshape(DIM, N_HASHES * (N_BUCKETS // 2))
    buckets = _hash_buckets(qk, rot128)

    bflat = buckets.reshape(B, N_HASHES * S)
    keys = bflat * S + (jnp.arange(N_HASHES * S, dtype=jnp.int32) % S)[None, :]
    sticker = jnp.argsort(keys, axis=-1).astype(jnp.int32)
    undo = jnp.argsort(sticker, axis=-1).astype(jnp.int32)
    st = sticker % S

    sqk = jnp.take_along_axis(qk, st[:, :, None], axis=1)
    sv = jnp.take_along_axis(v, st[:, :, None], axis=1)

    sqk4 = sqk.reshape(B, N_CHUNKS, CHUNK, DIM)
    sv4 = sv.reshape(B, N_CHUNKS, CHUNK, DIM)
    st4 = st.reshape(B, N_CHUNKS, 1, CHUNK)

    so4, slse4 = _attention(sqk4, sv4, st4)
    so = so4.reshape(B, N_HASHES * S, DIM)
    slse = slse4.reshape(B, N_HASHES * S)

    o = jnp.take_along_axis(so, undo[:, :, None], axis=1)
    lg = jnp.take_along_axis(slse, undo, axis=1)
    o = o.reshape(B, N_HASHES, S, DIM)
    lg = lg.reshape(B, N_HASHES, S)

    out = _combine(lg, o)
    attn = jnp.zeros((0,), dtype=qk.dtype)
    return out, attn, bflat


# speedup vs baseline: 3.0966x; 3.0966x over previous
import functools

import jax
import jax.numpy as jnp
from jax import lax
from jax.experimental import pallas as pl
from jax.experimental.pallas import tpu as pltpu
from jax.experimental.pallas import tpu_sc as plsc

B, S, DIM = 4, 4096, 64
N_HASHES = 4
BUCKET_SIZE = 64
N_BUCKETS = S // BUCKET_SIZE
N_CHUNKS = N_HASHES * N_BUCKETS
CHUNK = (N_HASHES * S) // N_CHUNKS
HS = N_HASHES * S
TOKEN_SELF_ATTN_VALUE = -5e4
NEG_BIG = 2**30

NC, NS = 2, 16
NW = NC * NS
ROWS_TOTAL = B * HS
ROWS_PER_W = ROWS_TOTAL // NW
GCHK = 128


def _hash_body(qk_ref, v_ref, rot_ref, buckets_ref, qkv_ref):
    x = qk_ref[0]
    qkv_ref[0] = jnp.concatenate([x, v_ref[0]], axis=1)
    r = jnp.dot(x, rot_ref[...], preferred_element_type=jnp.float32)
    iota = lax.broadcasted_iota(jnp.int32, (S, 2 * (N_BUCKETS // 2)), 1)
    for h in range(N_HASHES):
        seg = r[:, h * (N_BUCKETS // 2):(h + 1) * (N_BUCKETS // 2)]
        full = jnp.concatenate([seg, -seg], axis=1)
        m = jnp.max(full, axis=1, keepdims=True)
        idx = jnp.min(jnp.where(full == m, iota, NEG_BIG), axis=1)
        buckets_ref[0, h] = idx + h * N_BUCKETS


def _hash_buckets(qk, v, rot128):
    return pl.pallas_call(
        _hash_body,
        grid=(B,),
        in_specs=[
            pl.BlockSpec((1, S, DIM), lambda b: (b, 0, 0)),
            pl.BlockSpec((1, S, DIM), lambda b: (b, 0, 0)),
            pl.BlockSpec((DIM, N_HASHES * (N_BUCKETS // 2)), lambda b: (0, 0)),
        ],
        out_specs=[pl.BlockSpec((1, N_HASHES, S), lambda b: (b, 0, 0)),
                   pl.BlockSpec((1, S, 2 * DIM), lambda b: (b, 0, 0))],
        out_shape=[jax.ShapeDtypeStruct((B, N_HASHES, S), jnp.int32),
                   jax.ShapeDtypeStruct((B, S, 2 * DIM), jnp.float32)],
    )(qk, v, rot128)


def _sc_gather_rows(table2, idxg):
    mesh = plsc.VectorSubcoreMesh(core_axis_name="c", subcore_axis_name="s")

    @functools.partial(
        pl.kernel, mesh=mesh,
        out_type=jax.ShapeDtypeStruct((ROWS_TOTAL, 2 * DIM), jnp.float32),
        scratch_types=[pltpu.VMEM((GCHK,), jnp.int32),
                       pltpu.VMEM((GCHK, 2 * DIM), jnp.float32),
                       pltpu.SemaphoreType.DMA],
    )
    def k(tab_hbm, idx_hbm, out_hbm, idx_v, rows_v, sem):
        wid = lax.axis_index("s") * NC + lax.axis_index("c")

        def body(j, carry):
            base = wid * ROWS_PER_W + j * GCHK
            pltpu.sync_copy(idx_hbm.at[pl.ds(base, GCHK)], idx_v)
            pltpu.async_copy(tab_hbm.at[idx_v], rows_v, sem).wait()
            pltpu.sync_copy(rows_v, out_hbm.at[pl.ds(base, GCHK)])
            return carry

        lax.fori_loop(0, ROWS_PER_W // GCHK, body, 0)

    return k(table2, idxg)


def _attn_body(q_ref, kprev_ref, tq_ref, tprev_ref, so_ref):
    q = q_ref[0, 0][:, :DIM]
    kp = kprev_ref[0, 0][:, :DIM]

    def norm(x):
        n = jnp.sqrt(jnp.sum(x * x, axis=1, keepdims=True))
        return x / jnp.maximum(n, 1e-12)

    ks = norm(q)
    kpn = norm(kp)
    tq = tq_ref[0, 0, 0]
    tp = tprev_ref[0, 0, 0]

    scale = DIM ** -0.5
    d_self = lax.dot_general(q, ks, (((1,), (1,)), ((), ()))) * scale
    d_prev = lax.dot_general(q, kpn, (((1,), (1,)), ((), ()))) * scale
    d_self = jnp.where(tq[:, None] == tq[None, :], TOKEN_SELF_ATTN_VALUE, d_self)
    d_prev = jnp.where(tq[:, None] == tp[None, :], TOKEN_SELF_ATTN_VALUE, d_prev)
    dots = jnp.concatenate([d_self, d_prev], axis=1)

    m = jnp.max(dots, axis=1, keepdims=True)
    e = jnp.exp(dots - m)
    lse = m + jnp.log(jnp.sum(e, axis=1, keepdims=True))
    p = jnp.exp(dots - lse)
    vcat = jnp.concatenate([q_ref[0, 0][:, DIM:], kprev_ref[0, 0][:, DIM:]],
                           axis=0)
    o = jnp.dot(p, vcat, preferred_element_type=jnp.float32)
    pad = jnp.zeros((CHUNK, DIM - 1), dtype=jnp.float32)
    so_ref[0, 0] = jnp.concatenate([o, lse, pad], axis=1)


def _attention(sqkv4, st4):
    qb = pl.BlockSpec((1, 1, CHUNK, 2 * DIM), lambda b, c: (b, c, 0, 0))
    pb = pl.BlockSpec((1, 1, CHUNK, 2 * DIM),
                      lambda b, c: (b, (c - 1) % N_CHUNKS, 0, 0))
    tb = pl.BlockSpec((1, 1, 1, CHUNK), lambda b, c: (b, c, 0, 0))
    tpb = pl.BlockSpec((1, 1, 1, CHUNK),
                       lambda b, c: (b, (c - 1) % N_CHUNKS, 0, 0))
    return pl.pallas_call(
        _attn_body,
        grid=(B, N_CHUNKS),
        in_specs=[qb, pb, tb, tpb],
        out_specs=qb,
        out_shape=jax.ShapeDtypeStruct((B, N_CHUNKS, CHUNK, 2 * DIM),
                                       jnp.float32),
    )(sqkv4, sqkv4, st4, st4)


def _combine_body(o128_ref, out_ref):
    o128 = o128_ref[0]
    lg = o128[:, :, DIM]
    m = jnp.max(lg, axis=0, keepdims=True)
    lse = m + jnp.log(jnp.sum(jnp.exp(lg - m), axis=0, keepdims=True))
    w = jnp.exp(lg - lse)
    out_ref[0] = jnp.sum(o128[:, :, :DIM] * w[:, :, None], axis=0)


def _combine(o128):
    TBLK = 512
    return pl.pallas_call(
        _combine_body,
        grid=(B, S // TBLK),
        in_specs=[
            pl.BlockSpec((1, N_HASHES, TBLK, 2 * DIM),
                         lambda b, t: (b, 0, t, 0)),
        ],
        out_specs=pl.BlockSpec((1, TBLK, DIM), lambda b, t: (b, t, 0)),
        out_shape=jax.ShapeDtypeStruct((B, S, DIM), jnp.float32),
    )(o128)


def kernel(qk, v, random_rotations):
    rot128 = random_rotations[0].reshape(DIM, N_HASHES * (N_BUCKETS // 2))
    buckets, qkv = _hash_buckets(qk, v, rot128)

    bflat = buckets.reshape(B, HS)
    keys = bflat * S + (jnp.arange(HS, dtype=jnp.int32) % S)[None, :]
    sticker = jnp.argsort(keys, axis=-1).astype(jnp.int32)
    undo = jnp.argsort(sticker, axis=-1).astype(jnp.int32)
    st = sticker % S

    boff = (jnp.arange(B, dtype=jnp.int32) * S)[:, None]
    idxg = (st + boff).reshape(ROWS_TOTAL)
    sqkv2 = _sc_gather_rows(qkv.reshape(B * S, 2 * DIM), idxg)

    sqkv4 = sqkv2.reshape(B, N_CHUNKS, CHUNK, 2 * DIM)
    st4 = st.reshape(B, N_CHUNKS, 1, CHUNK)

    so128 = _attention(sqkv4, st4)

    uoff = (jnp.arange(B, dtype=jnp.int32) * HS)[:, None]
    undog = (undo + uoff).reshape(ROWS_TOTAL)
    o128 = _sc_gather_rows(so128.reshape(ROWS_TOTAL, 2 * DIM), undog)

    out = _combine(o128.reshape(B, N_HASHES, S, 2 * DIM))
    attn = jnp.zeros((0,), dtype=qk.dtype)
    return out, attn, bflat


# speedup vs baseline: 4.9318x; 1.5926x over previous
import functools

import jax
import jax.numpy as jnp
from jax import lax
from jax.experimental import pallas as pl
from jax.experimental.pallas import tpu as pltpu
from jax.experimental.pallas import tpu_sc as plsc

B, S, DIM = 4, 4096, 64
N_HASHES = 4
BUCKET_SIZE = 64
N_BUCKETS = S // BUCKET_SIZE
N_CHUNKS = N_HASHES * N_BUCKETS
CHUNK = (N_HASHES * S) // N_CHUNKS
HS = N_HASHES * S
TOKEN_SELF_ATTN_VALUE = -5e4
NEG_BIG = 2**30

NC, NS = 2, 16
NW = NC * NS
ROWS_TOTAL = B * HS
ROWS_PER_W = ROWS_TOTAL // NW
GCHK = 128


def _hash_body(qk_ref, v_ref, rot_ref, buckets_ref, qkv_ref):
    x = qk_ref[0]
    qkv_ref[0] = jnp.concatenate([x, v_ref[0]], axis=1)
    r = jnp.dot(x, rot_ref[...], preferred_element_type=jnp.float32)
    iota = lax.broadcasted_iota(jnp.int32, (S, 2 * (N_BUCKETS // 2)), 1)
    for h in range(N_HASHES):
        seg = r[:, h * (N_BUCKETS // 2):(h + 1) * (N_BUCKETS // 2)]
        full = jnp.concatenate([seg, -seg], axis=1)
        m = jnp.max(full, axis=1, keepdims=True)
        idx = jnp.min(jnp.where(full == m, iota, NEG_BIG), axis=1)
        buckets_ref[0, h] = idx + h * N_BUCKETS


def _hash_buckets(qk, v, rot128):
    return pl.pallas_call(
        _hash_body,
        grid=(B,),
        in_specs=[
            pl.BlockSpec((1, S, DIM), lambda b: (b, 0, 0)),
            pl.BlockSpec((1, S, DIM), lambda b: (b, 0, 0)),
            pl.BlockSpec((DIM, N_HASHES * (N_BUCKETS // 2)), lambda b: (0, 0)),
        ],
        out_specs=[pl.BlockSpec((1, N_HASHES, S), lambda b: (b, 0, 0)),
                   pl.BlockSpec((1, S, 2 * DIM), lambda b: (b, 0, 0))],
        out_shape=[jax.ShapeDtypeStruct((B, N_HASHES, S), jnp.int32),
                   jax.ShapeDtypeStruct((B, S, 2 * DIM), jnp.float32)],
    )(qk, v, rot128)


def _sc_gather_rows(table2, idxg):
    mesh = plsc.VectorSubcoreMesh(core_axis_name="c", subcore_axis_name="s")

    @functools.partial(
        pl.kernel, mesh=mesh,
        out_type=jax.ShapeDtypeStruct((ROWS_TOTAL, 2 * DIM), jnp.float32),
        scratch_types=[pltpu.VMEM((GCHK,), jnp.int32),
                       pltpu.VMEM((GCHK, 2 * DIM), jnp.float32),
                       pltpu.SemaphoreType.DMA],
    )
    def k(tab_hbm, idx_hbm, out_hbm, idx_v, rows_v, sem):
        wid = lax.axis_index("s") * NC + lax.axis_index("c")

        def body(j, carry):
            base = wid * ROWS_PER_W + j * GCHK
            pltpu.sync_copy(idx_hbm.at[pl.ds(base, GCHK)], idx_v)
            pltpu.async_copy(tab_hbm.at[idx_v], rows_v, sem).wait()
            pltpu.sync_copy(rows_v, out_hbm.at[pl.ds(base, GCHK)])
            return carry

        lax.fori_loop(0, ROWS_PER_W // GCHK, body, 0)

    return k(table2, idxg)


CB = 8
NB = N_CHUNKS // CB


def _attn_body(q_ref, kprev_ref, tq_ref, tprev_ref, so_ref):
    def norm(x):
        n = jnp.sqrt(jnp.sum(x * x, axis=1, keepdims=True))
        return x / jnp.maximum(n, 1e-12)

    scale = DIM ** -0.5
    kn = [norm(q_ref[0, i][:, :DIM]) for i in range(CB)]
    kn_last_prev = norm(kprev_ref[0, CB - 1][:, :DIM])
    for i in range(CB):
        q = q_ref[0, i][:, :DIM]
        kpn = kn[i - 1] if i > 0 else kn_last_prev
        pv = (q_ref[0, i - 1] if i > 0 else kprev_ref[0, CB - 1])[:, DIM:]
        tq = tq_ref[0, i, 0]
        tp = (tq_ref if i > 0 else tprev_ref)[0, i - 1 if i > 0 else CB - 1, 0]

        kcat = jnp.concatenate([kn[i], kpn], axis=0)
        tcat = jnp.concatenate([tq, tp], axis=0)
        dots = lax.dot_general(q, kcat, (((1,), (1,)), ((), ()))) * scale
        dots = jnp.where(tq[:, None] == tcat[None, :],
                         TOKEN_SELF_ATTN_VALUE, dots)

        m = jnp.max(dots, axis=1, keepdims=True)
        e = jnp.exp(dots - m)
        lse = m + jnp.log(jnp.sum(e, axis=1, keepdims=True))
        p = jnp.exp(dots - lse)
        vcat = jnp.concatenate([q_ref[0, i][:, DIM:], pv], axis=0)
        o = jnp.dot(p, vcat, preferred_element_type=jnp.float32)
        pad = jnp.zeros((CHUNK, DIM - 1), dtype=jnp.float32)
        so_ref[0, i] = jnp.concatenate([o, lse, pad], axis=1)


def _attention(sqkv4, st4):
    qb = pl.BlockSpec((1, CB, CHUNK, 2 * DIM), lambda b, c: (b, c, 0, 0))
    pb = pl.BlockSpec((1, CB, CHUNK, 2 * DIM),
                      lambda b, c: (b, (c - 1) % NB, 0, 0))
    tb = pl.BlockSpec((1, CB, 1, CHUNK), lambda b, c: (b, c, 0, 0))
    tpb = pl.BlockSpec((1, CB, 1, CHUNK),
                       lambda b, c: (b, (c - 1) % NB, 0, 0))
    return pl.pallas_call(
        _attn_body,
        grid=(B, NB),
        in_specs=[qb, pb, tb, tpb],
        out_specs=qb,
        out_shape=jax.ShapeDtypeStruct((B, N_CHUNKS, CHUNK, 2 * DIM),
                                       jnp.float32),
    )(sqkv4, sqkv4, st4, st4)


def _combine_body(o128_ref, out_ref):
    o128 = o128_ref[0]
    lg = o128[:, :, DIM]
    m = jnp.max(lg, axis=0, keepdims=True)
    lse = m + jnp.log(jnp.sum(jnp.exp(lg - m), axis=0, keepdims=True))
    w = jnp.exp(lg - lse)
    out_ref[0] = jnp.sum(o128[:, :, :DIM] * w[:, :, None], axis=0)


def _combine(o128):
    TBLK = 512
    return pl.pallas_call(
        _combine_body,
        grid=(B, S // TBLK),
        in_specs=[
            pl.BlockSpec((1, N_HASHES, TBLK, 2 * DIM),
                         lambda b, t: (b, 0, t, 0)),
        ],
        out_specs=pl.BlockSpec((1, TBLK, DIM), lambda b, t: (b, t, 0)),
        out_shape=jax.ShapeDtypeStruct((B, S, DIM), jnp.float32),
    )(o128)


def kernel(qk, v, random_rotations):
    rot128 = random_rotations[0].reshape(DIM, N_HASHES * (N_BUCKETS // 2))
    buckets, qkv = _hash_buckets(qk, v, rot128)

    bflat = buckets.reshape(B, HS)
    keys = bflat * S + (jnp.arange(HS, dtype=jnp.int32) % S)[None, :]
    sticker = jnp.argsort(keys, axis=-1).astype(jnp.int32)
    undo = jnp.argsort(sticker, axis=-1).astype(jnp.int32)
    st = sticker % S

    boff = (jnp.arange(B, dtype=jnp.int32) * S)[:, None]
    idxg = (st + boff).reshape(ROWS_TOTAL)
    sqkv2 = _sc_gather_rows(qkv.reshape(B * S, 2 * DIM), idxg)

    sqkv4 = sqkv2.reshape(B, N_CHUNKS, CHUNK, 2 * DIM)
    st4 = st.reshape(B, N_CHUNKS, 1, CHUNK)

    so128 = _attention(sqkv4, st4)

    uoff = (jnp.arange(B, dtype=jnp.int32) * HS)[:, None]
    undog = (undo + uoff).reshape(ROWS_TOTAL)
    o128 = _sc_gather_rows(so128.reshape(ROWS_TOTAL, 2 * DIM), undog)

    out = _combine(o128.reshape(B, N_HASHES, S, 2 * DIM))
    attn = jnp.zeros((0,), dtype=qk.dtype)
    return out, attn, bflat


# speedup vs baseline: 5.7145x; 1.1587x over previous
import functools

import jax
import jax.numpy as jnp
from jax import lax
from jax.experimental import pallas as pl
from jax.experimental.pallas import tpu as pltpu
from jax.experimental.pallas import tpu_sc as plsc

B, S, DIM = 4, 4096, 64
N_HASHES = 4
BUCKET_SIZE = 64
N_BUCKETS = S // BUCKET_SIZE
N_CHUNKS = N_HASHES * N_BUCKETS
CHUNK = (N_HASHES * S) // N_CHUNKS
HS = N_HASHES * S
TOKEN_SELF_ATTN_VALUE = -5e4
NEG_BIG = 2**30

NC, NS = 2, 16
NW = NC * NS
ROWS_TOTAL = B * HS
ROWS_PER_W = ROWS_TOTAL // NW
GCHK = 128


def _hash_body(qk_ref, v_ref, rot_ref, buckets_ref, qkv_ref):
    x = qk_ref[0]
    qkv_ref[0] = jnp.concatenate([x, v_ref[0]], axis=1)
    r = jnp.dot(x, rot_ref[...], preferred_element_type=jnp.float32)
    iota = lax.broadcasted_iota(jnp.int32, (S, 2 * (N_BUCKETS // 2)), 1)
    for h in range(N_HASHES):
        seg = r[:, h * (N_BUCKETS // 2):(h + 1) * (N_BUCKETS // 2)]
        full = jnp.concatenate([seg, -seg], axis=1)
        m = jnp.max(full, axis=1, keepdims=True)
        idx = jnp.min(jnp.where(full == m, iota, NEG_BIG), axis=1)
        buckets_ref[0, h] = idx + h * N_BUCKETS


def _hash_buckets(qk, v, rot128):
    return pl.pallas_call(
        _hash_body,
        grid=(B,),
        in_specs=[
            pl.BlockSpec((1, S, DIM), lambda b: (b, 0, 0)),
            pl.BlockSpec((1, S, DIM), lambda b: (b, 0, 0)),
            pl.BlockSpec((DIM, N_HASHES * (N_BUCKETS // 2)), lambda b: (0, 0)),
        ],
        out_specs=[pl.BlockSpec((1, N_HASHES, S), lambda b: (b, 0, 0)),
                   pl.BlockSpec((1, S, 2 * DIM), lambda b: (b, 0, 0))],
        out_shape=[jax.ShapeDtypeStruct((B, N_HASHES, S), jnp.int32),
                   jax.ShapeDtypeStruct((B, S, 2 * DIM), jnp.float32)],
    )(qk, v, rot128)


def _sc_bucket_sort(buckets):
    mesh = plsc.VectorSubcoreMesh(core_axis_name="c", subcore_axis_name="s")

    NVR = S // 16

    @functools.partial(
        pl.kernel, mesh=mesh,
        compiler_params=pltpu.CompilerParams(needs_layout_passes=False),
        out_type=[jax.ShapeDtypeStruct((B, HS), jnp.int32),
                  jax.ShapeDtypeStruct((B, HS), jnp.int32)],
        scratch_types=[pltpu.VMEM((S,), jnp.int32),
                       pltpu.VMEM((S,), jnp.int32),
                       pltpu.VMEM((S,), jnp.int32),
                       pltpu.VMEM((S,), jnp.int32),
                       pltpu.VMEM((N_BUCKETS * 16,), jnp.int32),
                       pltpu.VMEM((N_BUCKETS * 16,), jnp.int32)],
    )
    def k(bk_hbm, st_hbm, undo_hbm, bk_v, rank_v, st_v, und_v, hist_v, off_v):
        wid = lax.axis_index("s") * NC + lax.axis_index("c")

        @pl.when(wid < B * N_HASHES)
        def _():
            b = wid // N_HASHES
            h = wid % N_HASHES
            pltpu.sync_copy(bk_hbm.at[b, h], bk_v)
            hoff = h * N_BUCKETS
            lane = lax.iota(jnp.int32, 16)
            zeros = jnp.zeros((16,), jnp.int32)

            def zero(i, c):
                hist_v[pl.ds(i * 16, 16)] = zeros
                return c

            lax.fori_loop(0, N_BUCKETS, zero, 0)

            def count(i, c):
                s16 = lane * NVR + i
                bkt = plsc.load_gather(bk_v, [s16]) - hoff
                slot = bkt * 16 + lane
                r = plsc.load_gather(hist_v, [slot])
                rank_v[pl.ds(i * 16, 16)] = r
                plsc.store_scatter(hist_v, [slot], r + 1)
                return c

            lax.fori_loop(0, NVR, count, 0)

            def prefix(kk, run):
                row = hist_v[pl.ds(kk * 16, 16)]
                csum = plsc.cumsum(row)
                off_v[pl.ds(kk * 16, 16)] = (csum - row) + run
                return run + jnp.sum(row)

            lax.fori_loop(0, N_BUCKETS, prefix, 0)

            def place(i, c):
                s16 = lane * NVR + i
                bkt = plsc.load_gather(bk_v, [s16]) - hoff
                slot = bkt * 16 + lane
                base = plsc.load_gather(off_v, [slot])
                pos16 = base + rank_v[pl.ds(i * 16, 16)]
                plsc.store_scatter(st_v, [pos16], s16)
                plsc.store_scatter(und_v, [s16], pos16 + h * S)
                return c

            lax.fori_loop(0, NVR, place, 0)
            pltpu.sync_copy(st_v, st_hbm.at[b, pl.ds(h * S, S)])
            pltpu.sync_copy(und_v, undo_hbm.at[b, pl.ds(h * S, S)])

    return k(buckets)


def _sc_gather_rows(table2, idxg):
    mesh = plsc.VectorSubcoreMesh(core_axis_name="c", subcore_axis_name="s")

    @functools.partial(
        pl.kernel, mesh=mesh,
        out_type=jax.ShapeDtypeStruct((ROWS_TOTAL, 2 * DIM), jnp.float32),
        scratch_types=[pltpu.VMEM((GCHK,), jnp.int32),
                       pltpu.VMEM((GCHK, 2 * DIM), jnp.float32),
                       pltpu.SemaphoreType.DMA],
    )
    def k(tab_hbm, idx_hbm, out_hbm, idx_v, rows_v, sem):
        wid = lax.axis_index("s") * NC + lax.axis_index("c")

        def body(j, carry):
            base = wid * ROWS_PER_W + j * GCHK
            pltpu.sync_copy(idx_hbm.at[pl.ds(base, GCHK)], idx_v)
            pltpu.async_copy(tab_hbm.at[idx_v], rows_v, sem).wait()
            pltpu.sync_copy(rows_v, out_hbm.at[pl.ds(base, GCHK)])
            return carry

        lax.fori_loop(0, ROWS_PER_W // GCHK, body, 0)

    return k(table2, idxg)


CB = 8
NB = N_CHUNKS // CB


def _attn_body(q_ref, kprev_ref, tq_ref, tprev_ref, so_ref):
    def norm(x):
        n = jnp.sqrt(jnp.sum(x * x, axis=1, keepdims=True))
        return x / jnp.maximum(n, 1e-12)

    scale = DIM ** -0.5
    kn = [norm(q_ref[0, i][:, :DIM]) for i in range(CB)]
    kn_last_prev = norm(kprev_ref[0, CB - 1][:, :DIM])
    for i in range(CB):
        q = q_ref[0, i][:, :DIM]
        kpn = kn[i - 1] if i > 0 else kn_last_prev
        pv = (q_ref[0, i - 1] if i > 0 else kprev_ref[0, CB - 1])[:, DIM:]
        tq = tq_ref[0, i, 0]
        tp = (tq_ref if i > 0 else tprev_ref)[0, i - 1 if i > 0 else CB - 1, 0]

        kcat = jnp.concatenate([kn[i], kpn], axis=0)
        tcat = jnp.concatenate([tq, tp], axis=0)
        dots = lax.dot_general(q, kcat, (((1,), (1,)), ((), ()))) * scale
        dots = jnp.where(tq[:, None] == tcat[None, :],
                         TOKEN_SELF_ATTN_VALUE, dots)

        m = jnp.max(dots, axis=1, keepdims=True)
        e = jnp.exp(dots - m)
        ssum = jnp.sum(e, axis=1, keepdims=True)
        lse = m + jnp.log(ssum)
        p = e * (1.0 / ssum)
        vcat = jnp.concatenate([q_ref[0, i][:, DIM:], pv], axis=0)
        o = jnp.dot(p, vcat, preferred_element_type=jnp.float32)
        pad = jnp.zeros((CHUNK, DIM - 1), dtype=jnp.float32)
        so_ref[0, i] = jnp.concatenate([o, lse, pad], axis=1)


def _attention(sqkv4, st4):
    qb = pl.BlockSpec((1, CB, CHUNK, 2 * DIM), lambda b, c: (b, c, 0, 0))
    pb = pl.BlockSpec((1, CB, CHUNK, 2 * DIM),
                      lambda b, c: (b, (c - 1) % NB, 0, 0))
    tb = pl.BlockSpec((1, CB, 1, CHUNK), lambda b, c: (b, c, 0, 0))
    tpb = pl.BlockSpec((1, CB, 1, CHUNK),
                       lambda b, c: (b, (c - 1) % NB, 0, 0))
    return pl.pallas_call(
        _attn_body,
        grid=(B, NB),
        in_specs=[qb, pb, tb, tpb],
        out_specs=qb,
        out_shape=jax.ShapeDtypeStruct((B, N_CHUNKS, CHUNK, 2 * DIM),
                                       jnp.float32),
    )(sqkv4, sqkv4, st4, st4)


def _combine_body(o128_ref, out_ref):
    o128 = o128_ref[0]
    lg = o128[:, :, DIM]
    m = jnp.max(lg, axis=0, keepdims=True)
    lse = m + jnp.log(jnp.sum(jnp.exp(lg - m), axis=0, keepdims=True))
    w = jnp.exp(lg - lse)
    out_ref[0] = jnp.sum(o128[:, :, :DIM] * w[:, :, None], axis=0)


def _combine(o128):
    TBLK = 512
    return pl.pallas_call(
        _combine_body,
        grid=(B, S // TBLK),
        in_specs=[
            pl.BlockSpec((1, N_HASHES, TBLK, 2 * DIM),
                         lambda b, t: (b, 0, t, 0)),
        ],
        out_specs=pl.BlockSpec((1, TBLK, DIM), lambda b, t: (b, t, 0)),
        out_shape=jax.ShapeDtypeStruct((B, S, DIM), jnp.float32),
    )(o128)


def kernel(qk, v, random_rotations):
    rot128 = random_rotations[0].reshape(DIM, N_HASHES * (N_BUCKETS // 2))
    buckets, qkv = _hash_buckets(qk, v, rot128)

    bflat = buckets.reshape(B, HS)
    st, undo = _sc_bucket_sort(buckets)

    boff = (jnp.arange(B, dtype=jnp.int32) * S)[:, None]
    idxg = (st + boff).reshape(ROWS_TOTAL)
    sqkv2 = _sc_gather_rows(qkv.reshape(B * S, 2 * DIM), idxg)

    sqkv4 = sqkv2.reshape(B, N_CHUNKS, CHUNK, 2 * DIM)
    st4 = st.reshape(B, N_CHUNKS, 1, CHUNK)

    so128 = _attention(sqkv4, st4)

    uoff = (jnp.arange(B, dtype=jnp.int32) * HS)[:, None]
    undog = (undo + uoff).reshape(ROWS_TOTAL)
    o128 = _sc_gather_rows(so128.reshape(ROWS_TOTAL, 2 * DIM), undog)

    out = _combine(o128.reshape(B, N_HASHES, S, 2 * DIM))
    attn = jnp.zeros((0,), dtype=qk.dtype)
    return out, attn, bflat


# speedup vs baseline: 5.7179x; 1.0006x over previous
import functools

import jax
import jax.numpy as jnp
from jax import lax
from jax.experimental import pallas as pl
from jax.experimental.pallas import tpu as pltpu
from jax.experimental.pallas import tpu_sc as plsc

B, S, DIM = 4, 4096, 64
N_HASHES = 4
BUCKET_SIZE = 64
N_BUCKETS = S // BUCKET_SIZE
N_CHUNKS = N_HASHES * N_BUCKETS
CHUNK = (N_HASHES * S) // N_CHUNKS
HS = N_HASHES * S
TOKEN_SELF_ATTN_VALUE = -5e4
NEG_BIG = 2**30

NC, NS = 2, 16
NW = NC * NS
ROWS_TOTAL = B * HS
ROWS_PER_W = ROWS_TOTAL // NW
GCHK = 128


def _hash_body(qk_ref, v_ref, rot_ref, buckets_ref, qkv_ref):
    x = qk_ref[0]
    qkv_ref[0] = jnp.concatenate([x, v_ref[0]], axis=1)
    r = jnp.dot(x, rot_ref[...], preferred_element_type=jnp.float32)
    iota = lax.broadcasted_iota(jnp.int32, (S, 2 * (N_BUCKETS // 2)), 1)
    for h in range(N_HASHES):
        seg = r[:, h * (N_BUCKETS // 2):(h + 1) * (N_BUCKETS // 2)]
        full = jnp.concatenate([seg, -seg], axis=1)
        m = jnp.max(full, axis=1, keepdims=True)
        idx = jnp.min(jnp.where(full == m, iota, NEG_BIG), axis=1)
        buckets_ref[0, h] = idx + h * N_BUCKETS


def _hash_buckets(qk, v, rot128):
    return pl.pallas_call(
        _hash_body,
        grid=(B,),
        in_specs=[
            pl.BlockSpec((1, S, DIM), lambda b: (b, 0, 0)),
            pl.BlockSpec((1, S, DIM), lambda b: (b, 0, 0)),
            pl.BlockSpec((DIM, N_HASHES * (N_BUCKETS // 2)), lambda b: (0, 0)),
        ],
        out_specs=[pl.BlockSpec((1, N_HASHES, S), lambda b: (b, 0, 0)),
                   pl.BlockSpec((1, S, 2 * DIM), lambda b: (b, 0, 0))],
        out_shape=[jax.ShapeDtypeStruct((B, N_HASHES, S), jnp.int32),
                   jax.ShapeDtypeStruct((B, S, 2 * DIM), jnp.float32)],
    )(qk, v, rot128)


def _sc_bucket_sort(buckets):
    mesh = plsc.VectorSubcoreMesh(core_axis_name="c", subcore_axis_name="s")

    NVR = S // 16

    @functools.partial(
        pl.kernel, mesh=mesh,
        compiler_params=pltpu.CompilerParams(needs_layout_passes=False),
        out_type=[jax.ShapeDtypeStruct((B, HS), jnp.int32),
                  jax.ShapeDtypeStruct((B, HS), jnp.int32)],
        scratch_types=[pltpu.VMEM((S,), jnp.int32),
                       pltpu.VMEM((S,), jnp.int32),
                       pltpu.VMEM((S,), jnp.int32),
                       pltpu.VMEM((S,), jnp.int32),
                       pltpu.VMEM((N_BUCKETS * 16,), jnp.int32),
                       pltpu.VMEM((N_BUCKETS * 16,), jnp.int32)],
    )
    def k(bk_hbm, st_hbm, undo_hbm, bk_v, rank_v, st_v, und_v, hist_v, off_v):
        wid = lax.axis_index("s") * NC + lax.axis_index("c")

        @pl.when(wid < B * N_HASHES)
        def _():
            b = wid // N_HASHES
            h = wid % N_HASHES
            pltpu.sync_copy(bk_hbm.at[b, h], bk_v)
            hoff = h * N_BUCKETS
            lane = lax.iota(jnp.int32, 16)
            zeros = jnp.zeros((16,), jnp.int32)

            def zero(i, c):
                hist_v[pl.ds(i * 16, 16)] = zeros
                return c

            lax.fori_loop(0, N_BUCKETS, zero, 0)

            def count(i, c):
                s16 = lane * NVR + i
                bkt = plsc.load_gather(bk_v, [s16]) - hoff
                slot = bkt * 16 + lane
                r = plsc.load_gather(hist_v, [slot])
                rank_v[pl.ds(i * 16, 16)] = r
                plsc.store_scatter(hist_v, [slot], r + 1)
                return c

            lax.fori_loop(0, NVR, count, 0)

            def prefix(kk, run):
                row = hist_v[pl.ds(kk * 16, 16)]
                csum = plsc.cumsum(row)
                off_v[pl.ds(kk * 16, 16)] = (csum - row) + run
                return run + jnp.sum(row)

            lax.fori_loop(0, N_BUCKETS, prefix, 0)

            def place(i, c):
                s16 = lane * NVR + i
                bkt = plsc.load_gather(bk_v, [s16]) - hoff
                slot = bkt * 16 + lane
                base = plsc.load_gather(off_v, [slot])
                pos16 = base + rank_v[pl.ds(i * 16, 16)]
                plsc.store_scatter(st_v, [pos16], s16)
                plsc.store_scatter(und_v, [s16], pos16 + h * S)
                return c

            lax.fori_loop(0, NVR, place, 0)
            pltpu.sync_copy(st_v, st_hbm.at[b, pl.ds(h * S, S)])
            pltpu.sync_copy(und_v, undo_hbm.at[b, pl.ds(h * S, S)])

    return k(buckets)


def _sc_gather_rows(table2, idxg):
    mesh = plsc.VectorSubcoreMesh(core_axis_name="c", subcore_axis_name="s")

    @functools.partial(
        pl.kernel, mesh=mesh,
        out_type=jax.ShapeDtypeStruct((ROWS_TOTAL, 2 * DIM), jnp.float32),
        scratch_types=[pltpu.VMEM((GCHK,), jnp.int32),
                       pltpu.VMEM((GCHK, 2 * DIM), jnp.float32),
                       pltpu.SemaphoreType.DMA],
    )
    def k(tab_hbm, idx_hbm, out_hbm, idx_v, rows_v, sem):
        wid = lax.axis_index("s") * NC + lax.axis_index("c")

        def body(j, carry):
            base = wid * ROWS_PER_W + j * GCHK
            pltpu.sync_copy(idx_hbm.at[pl.ds(base, GCHK)], idx_v)
            pltpu.async_copy(tab_hbm.at[idx_v], rows_v, sem).wait()
            pltpu.sync_copy(rows_v, out_hbm.at[pl.ds(base, GCHK)])
            return carry

        lax.fori_loop(0, ROWS_PER_W // GCHK, body, 0)

    return k(table2, idxg)


CB = 8
NB = N_CHUNKS // CB


def _attn_body(q_ref, kprev_ref, tq_ref, tprev_ref, so_ref):
    def norm(x):
        n = jnp.sqrt(jnp.sum(x * x, axis=1, keepdims=True))
        return x / jnp.maximum(n, 1e-12)

    scale = DIM ** -0.5
    kn = [norm(q_ref[0, i][:, :DIM]) for i in range(CB)]
    kn_last_prev = norm(kprev_ref[0, 0][:, :DIM])
    for i in range(CB):
        q = q_ref[0, i][:, :DIM]
        kpn = kn[i - 1] if i > 0 else kn_last_prev
        pv = (q_ref[0, i - 1] if i > 0 else kprev_ref[0, 0])[:, DIM:]
        tq = tq_ref[0, i, 0]
        tp = tq_ref[0, i - 1, 0] if i > 0 else tprev_ref[0, 0, 0]

        kcat = jnp.concatenate([kn[i], kpn], axis=0)
        tcat = jnp.concatenate([tq, tp], axis=0)
        dots = lax.dot_general(q, kcat, (((1,), (1,)), ((), ()))) * scale
        dots = jnp.where(tq[:, None] == tcat[None, :],
                         TOKEN_SELF_ATTN_VALUE, dots)

        m = jnp.max(dots, axis=1, keepdims=True)
        e = jnp.exp(dots - m)
        ssum = jnp.sum(e, axis=1, keepdims=True)
        lse = m + jnp.log(ssum)
        p = e * (1.0 / ssum)
        vcat = jnp.concatenate([q_ref[0, i][:, DIM:], pv], axis=0)
        o = jnp.dot(p, vcat, preferred_element_type=jnp.float32)
        pad = jnp.zeros((CHUNK, DIM - 1), dtype=jnp.float32)
        so_ref[0, i] = jnp.concatenate([o, lse, pad], axis=1)


def _attention(sqkv4, st4):
    qb = pl.BlockSpec((1, CB, CHUNK, 2 * DIM), lambda b, c: (b, c, 0, 0))
    pb = pl.BlockSpec((1, 1, CHUNK, 2 * DIM),
                      lambda b, c: (b, (c * CB - 1) % N_CHUNKS, 0, 0))
    tb = pl.BlockSpec((1, CB, 1, CHUNK), lambda b, c: (b, c, 0, 0))
    tpb = pl.BlockSpec((1, 1, 1, CHUNK),
                       lambda b, c: (b, (c * CB - 1) % N_CHUNKS, 0, 0))
    return pl.pallas_call(
        _attn_body,
        grid=(B, NB),
        in_specs=[qb, pb, tb, tpb],
        out_specs=qb,
        out_shape=jax.ShapeDtypeStruct((B, N_CHUNKS, CHUNK, 2 * DIM),
                                       jnp.float32),
    )(sqkv4, sqkv4, st4, st4)


def _combine_body(o128_ref, out_ref):
    o128 = o128_ref[0]
    lg = o128[:, :, DIM]
    m = jnp.max(lg, axis=0, keepdims=True)
    lse = m + jnp.log(jnp.sum(jnp.exp(lg - m), axis=0, keepdims=True))
    w = jnp.exp(lg - lse)
    out_ref[0] = jnp.sum(o128[:, :, :DIM] * w[:, :, None], axis=0)


def _combine(o128):
    TBLK = 512
    return pl.pallas_call(
        _combine_body,
        grid=(B, S // TBLK),
        in_specs=[
            pl.BlockSpec((1, N_HASHES, TBLK, 2 * DIM),
                         lambda b, t: (b, 0, t, 0)),
        ],
        out_specs=pl.BlockSpec((1, TBLK, DIM), lambda b, t: (b, t, 0)),
        out_shape=jax.ShapeDtypeStruct((B, S, DIM), jnp.float32),
    )(o128)


def kernel(qk, v, random_rotations):
    rot128 = random_rotations[0].reshape(DIM, N_HASHES * (N_BUCKETS // 2))
    buckets, qkv = _hash_buckets(qk, v, rot128)

    bflat = buckets.reshape(B, HS)
    st, undo = _sc_bucket_sort(buckets)

    boff = (jnp.arange(B, dtype=jnp.int32) * S)[:, None]
    idxg = (st + boff).reshape(ROWS_TOTAL)
    sqkv2 = _sc_gather_rows(qkv.reshape(B * S, 2 * DIM), idxg)

    sqkv4 = sqkv2.reshape(B, N_CHUNKS, CHUNK, 2 * DIM)
    st4 = st.reshape(B, N_CHUNKS, 1, CHUNK)

    so128 = _attention(sqkv4, st4)

    uoff = (jnp.arange(B, dtype=jnp.int32) * HS)[:, None]
    undog = (undo + uoff).reshape(ROWS_TOTAL)
    o128 = _sc_gather_rows(so128.reshape(ROWS_TOTAL, 2 * DIM), undog)

    out = _combine(o128.reshape(B, N_HASHES, S, 2 * DIM))
    attn = jnp.zeros((0,), dtype=qk.dtype)
    return out, attn, bflat


# speedup vs baseline: 8.9081x; 1.5579x over previous
import functools

import jax
import jax.numpy as jnp
from jax import lax
from jax.experimental import pallas as pl
from jax.experimental.pallas import tpu as pltpu
from jax.experimental.pallas import tpu_sc as plsc

B, S, DIM = 4, 4096, 64
N_HASHES = 4
BUCKET_SIZE = 64
N_BUCKETS = S // BUCKET_SIZE
N_CHUNKS = N_HASHES * N_BUCKETS
CHUNK = (N_HASHES * S) // N_CHUNKS
HS = N_HASHES * S
TOKEN_SELF_ATTN_VALUE = -5e4
NEG_BIG = 2**30

NC, NS = 2, 16
NW = NC * NS
ROWS_TOTAL = B * HS
ROWS_PER_W = ROWS_TOTAL // NW
GCHK = 128


def _hash_body(qk_ref, v_ref, rot_ref, buckets_ref, qkv_ref):
    x = qk_ref[0]
    qkv_ref[0] = jnp.concatenate([x, v_ref[0]], axis=1)
    r = jnp.dot(x, rot_ref[...], preferred_element_type=jnp.float32)
    iota = lax.broadcasted_iota(jnp.int32, (S, 2 * (N_BUCKETS // 2)), 1)
    for h in range(N_HASHES):
        seg = r[:, h * (N_BUCKETS // 2):(h + 1) * (N_BUCKETS // 2)]
        full = jnp.concatenate([seg, -seg], axis=1)
        m = jnp.max(full, axis=1, keepdims=True)
        idx = jnp.min(jnp.where(full == m, iota, NEG_BIG), axis=1)
        buckets_ref[0, h] = idx + h * N_BUCKETS


def _hash_buckets(qk, v, rot128):
    return pl.pallas_call(
        _hash_body,
        grid=(B,),
        in_specs=[
            pl.BlockSpec((1, S, DIM), lambda b: (b, 0, 0)),
            pl.BlockSpec((1, S, DIM), lambda b: (b, 0, 0)),
            pl.BlockSpec((DIM, N_HASHES * (N_BUCKETS // 2)), lambda b: (0, 0)),
        ],
        out_specs=[pl.BlockSpec((1, N_HASHES, S), lambda b: (b, 0, 0)),
                   pl.BlockSpec((1, S, 2 * DIM), lambda b: (b, 0, 0))],
        out_shape=[jax.ShapeDtypeStruct((B, N_HASHES, S), jnp.int32),
                   jax.ShapeDtypeStruct((B, S, 2 * DIM), jnp.float32)],
    )(qk, v, rot128)


def _sc_bucket_sort(buckets):
    mesh = plsc.VectorSubcoreMesh(core_axis_name="c", subcore_axis_name="s")

    NVR = S // 16

    @functools.partial(
        pl.kernel, mesh=mesh,
        compiler_params=pltpu.CompilerParams(needs_layout_passes=False),
        out_type=[jax.ShapeDtypeStruct((B, HS), jnp.int32),
                  jax.ShapeDtypeStruct((B, HS), jnp.int32)],
        scratch_types=[pltpu.VMEM((S,), jnp.int32),
                       pltpu.VMEM((S,), jnp.int32),
                       pltpu.VMEM((S,), jnp.int32),
                       pltpu.VMEM((S,), jnp.int32),
                       pltpu.VMEM((N_BUCKETS * 16,), jnp.int32),
                       pltpu.VMEM((N_BUCKETS * 16,), jnp.int32)],
    )
    def k(bk_hbm, st_hbm, undo_hbm, bk_v, rank_v, st_v, und_v, hist_v, off_v):
        wid = lax.axis_index("s") * NC + lax.axis_index("c")

        @pl.when(wid < B * N_HASHES)
        def _():
            b = wid // N_HASHES
            h = wid % N_HASHES
            pltpu.sync_copy(bk_hbm.at[b, h], bk_v)
            hoff = h * N_BUCKETS
            lane = lax.iota(jnp.int32, 16)
            zeros = jnp.zeros((16,), jnp.int32)

            def zero(i, c):
                hist_v[pl.ds(i * 16, 16)] = zeros
                return c

            lax.fori_loop(0, N_BUCKETS, zero, 0)

            def count(i, c):
                s16 = lane * NVR + i
                bkt = plsc.load_gather(bk_v, [s16]) - hoff
                slot = bkt * 16 + lane
                r = plsc.load_gather(hist_v, [slot])
                rank_v[pl.ds(i * 16, 16)] = r
                plsc.store_scatter(hist_v, [slot], r + 1)
                return c

            lax.fori_loop(0, NVR, count, 0)

            def prefix(kk, run):
                row = hist_v[pl.ds(kk * 16, 16)]
                csum = plsc.cumsum(row)
                off_v[pl.ds(kk * 16, 16)] = (csum - row) + run
                return run + jnp.sum(row)

            lax.fori_loop(0, N_BUCKETS, prefix, 0)

            def place(i, c):
                s16 = lane * NVR + i
                bkt = plsc.load_gather(bk_v, [s16]) - hoff
                slot = bkt * 16 + lane
                base = plsc.load_gather(off_v, [slot])
                pos16 = base + rank_v[pl.ds(i * 16, 16)]
                plsc.store_scatter(st_v, [pos16], s16)
                plsc.store_scatter(und_v, [s16], pos16 + h * S)
                return c

            lax.fori_loop(0, NVR, place, 0)
            pltpu.sync_copy(st_v, st_hbm.at[b, pl.ds(h * S, S)])
            pltpu.sync_copy(und_v, undo_hbm.at[b, pl.ds(h * S, S)])

    return k(buckets)


def _sc_gather_rows(table2, idxg):
    mesh = plsc.VectorSubcoreMesh(core_axis_name="c", subcore_axis_name="s")

    @functools.partial(
        pl.kernel, mesh=mesh,
        out_type=jax.ShapeDtypeStruct((ROWS_TOTAL, 2 * DIM), jnp.float32),
        scratch_types=[pltpu.VMEM((GCHK,), jnp.int32),
                       pltpu.VMEM((GCHK, 2 * DIM), jnp.float32),
                       pltpu.SemaphoreType.DMA],
    )
    def k(tab_hbm, idx_hbm, out_hbm, idx_v, rows_v, sem):
        wid = lax.axis_index("s") * NC + lax.axis_index("c")

        def body(j, carry):
            base = wid * ROWS_PER_W + j * GCHK
            pltpu.sync_copy(idx_hbm.at[pl.ds(base, GCHK)], idx_v)
            pltpu.async_copy(tab_hbm.at[idx_v], rows_v, sem).wait()
            pltpu.sync_copy(rows_v, out_hbm.at[pl.ds(base, GCHK)])
            return carry

        lax.fori_loop(0, ROWS_PER_W // GCHK, body, 0)

    return k(table2, idxg)


CB = 8
NB = N_CHUNKS // CB


def _attn_body(q_ref, kprev_ref, tq_ref, tprev_ref, so_ref):
    R = CB * CHUNK
    flat = q_ref[0].reshape(R, 2 * DIM)
    qall = flat[:, :DIM]
    allrows = jnp.concatenate([kprev_ref[0, 0], flat], axis=0)
    aq = allrows[:, :DIM]
    av = allrows[:, DIM:]
    ssq = jnp.sum(aq * aq, axis=1, keepdims=True)
    akn = aq / jnp.maximum(jnp.sqrt(ssq), 1e-12)

    scale = DIM ** -0.5
    dots_l = []
    for i in range(CB):
        q = qall[i * CHUNK:(i + 1) * CHUNK]
        kwin = akn[i * CHUNK:(i + 2) * CHUNK]
        d = lax.dot_general(q, kwin, (((1,), (1,)), ((), ()))) * scale
        tqi = tq_ref[0, i, 0]
        tpi = tq_ref[0, i - 1, 0] if i > 0 else tprev_ref[0, 0, 0]
        twin = jnp.concatenate([tpi, tqi], axis=0)
        d = jnp.where(tqi[:, None] == twin[None, :], TOKEN_SELF_ATTN_VALUE, d)
        dots_l.append(d)
    dots = jnp.concatenate(dots_l, axis=0)

    m = jnp.max(dots, axis=1, keepdims=True)
    e = jnp.exp(dots - m)
    ssum = jnp.sum(e, axis=1, keepdims=True)
    lse = m + jnp.log(ssum)
    p = e * (1.0 / ssum)

    for i in range(CB):
        pi = p[i * CHUNK:(i + 1) * CHUNK]
        vwin = av[i * CHUNK:(i + 2) * CHUNK]
        o = jnp.dot(pi, vwin, preferred_element_type=jnp.float32)
        pad = jnp.zeros((CHUNK, DIM - 1), dtype=jnp.float32)
        so_ref[0, i] = jnp.concatenate(
            [o, lse[i * CHUNK:(i + 1) * CHUNK], pad], axis=1)


def _attention(sqkv4, st4):
    qb = pl.BlockSpec((1, CB, CHUNK, 2 * DIM), lambda b, c: (b, c, 0, 0))
    pb = pl.BlockSpec((1, 1, CHUNK, 2 * DIM),
                      lambda b, c: (b, (c * CB - 1) % N_CHUNKS, 0, 0))
    tb = pl.BlockSpec((1, CB, 1, CHUNK), lambda b, c: (b, c, 0, 0))
    tpb = pl.BlockSpec((1, 1, 1, CHUNK),
                       lambda b, c: (b, (c * CB - 1) % N_CHUNKS, 0, 0))
    return pl.pallas_call(
        _attn_body,
        grid=(B, NB),
        in_specs=[qb, pb, tb, tpb],
        out_specs=qb,
        out_shape=jax.ShapeDtypeStruct((B, N_CHUNKS, CHUNK, 2 * DIM),
                                       jnp.float32),
    )(sqkv4, sqkv4, st4, st4)


def _combine_body(o128_ref, out_ref):
    o128 = o128_ref[0]
    lg = o128[:, :, DIM]
    m = jnp.max(lg, axis=0, keepdims=True)
    lse = m + jnp.log(jnp.sum(jnp.exp(lg - m), axis=0, keepdims=True))
    w = jnp.exp(lg - lse)
    out_ref[0] = jnp.sum(o128[:, :, :DIM] * w[:, :, None], axis=0)


def _combine(o128):
    TBLK = 512
    return pl.pallas_call(
        _combine_body,
        grid=(B, S // TBLK),
        in_specs=[
            pl.BlockSpec((1, N_HASHES, TBLK, 2 * DIM),
                         lambda b, t: (b, 0, t, 0)),
        ],
        out_specs=pl.BlockSpec((1, TBLK, DIM), lambda b, t: (b, t, 0)),
        out_shape=jax.ShapeDtypeStruct((B, S, DIM), jnp.float32),
    )(o128)


def kernel(qk, v, random_rotations):
    rot128 = random_rotations[0].reshape(DIM, N_HASHES * (N_BUCKETS // 2))
    buckets, qkv = _hash_buckets(qk, v, rot128)

    bflat = buckets.reshape(B, HS)
    st, undo = _sc_bucket_sort(buckets)

    boff = (jnp.arange(B, dtype=jnp.int32) * S)[:, None]
    idxg = (st + boff).reshape(ROWS_TOTAL)
    sqkv2 = _sc_gather_rows(qkv.reshape(B * S, 2 * DIM), idxg)

    sqkv4 = sqkv2.reshape(B, N_CHUNKS, CHUNK, 2 * DIM)
    st4 = st.reshape(B, N_CHUNKS, 1, CHUNK)

    so128 = _attention(sqkv4, st4)

    uoff = (jnp.arange(B, dtype=jnp.int32) * HS)[:, None]
    undog = (undo + uoff).reshape(ROWS_TOTAL)
    o128 = _sc_gather_rows(so128.reshape(ROWS_TOTAL, 2 * DIM), undog)

    out = _combine(o128.reshape(B, N_HASHES, S, 2 * DIM))
    attn = jnp.zeros((0,), dtype=qk.dtype)
    return out, attn, bflat
